# baseline (speedup 1.0000x reference)
import jax
import jax.numpy as jnp
from jax.experimental import pallas as pl
from jax.experimental.pallas import tpu as pltpu

_INPUT_DIM = 768
_LATENT_DIM = 3072
_K = 32
_TM = 256


def _fused_body(xt_ref, wenc_ref, benc_ref, wdec_ref, bdec_ref,
                sparse_ref, recon_ref):
    lat_t = jax.lax.dot_general(
        wenc_ref[...], xt_ref[...], (((1,), (0,)), ((), ())),
        preferred_element_type=jnp.float32,
        precision=jax.lax.Precision.DEFAULT) + benc_ref[...]

    bits = jax.lax.bitcast_convert_type(lat_t, jnp.int32)
    keys = jnp.where(bits < 0, jnp.int32(-2147483648) - bits, bits)

    lo0 = jnp.full((1, _TM), -2139095041, jnp.int32)
    hi0 = jnp.full((1, _TM), 2139095041, jnp.int32)
    cnt0 = jnp.zeros((1, _TM), jnp.int32)

    def bs_body(_, carry):
        lo, hi, cnt_hi = carry
        mid = (lo >> 1) + (hi >> 1) + (lo & hi & 1)
        cnt = jnp.sum((keys > mid).astype(jnp.int32), axis=0, keepdims=True)
        ge_k = cnt >= _K
        lo = jnp.where(ge_k, mid + 1, lo)
        hi = jnp.where(ge_k, hi, mid)
        cnt_hi = jnp.where(ge_k, cnt_hi, cnt)
        return lo, hi, cnt_hi

    lo, _, cnt_gt = jax.lax.fori_loop(0, 32, bs_body, (lo0, hi0, cnt0),
                                      unroll=4)
    vstar = lo

    deficit = _K - cnt_gt
    eq = keys == vstar
    iota = jax.lax.broadcasted_iota(jnp.int32, keys.shape, 0)
    sparse0 = jnp.where(keys > vstar, lat_t, 0.0)

    def tie_body(_, carry):
        sparse, deficit = carry
        pick = jnp.logical_and(eq, sparse == 0.0)
        cand = jnp.where(pick, iota, _LATENT_DIM)
        amin = jnp.min(cand, axis=0, keepdims=True)
        add = jnp.logical_and(iota == amin, deficit > 0)
        sparse = jnp.where(add, lat_t, sparse)
        deficit = deficit - (deficit > 0).astype(jnp.int32)
        return sparse, deficit

    sparse_t, _ = jax.lax.fori_loop(0, 2, tie_body, (sparse0, deficit))
    sparse_ref[...] = sparse_t.T
    recon_t = jax.lax.dot_general(
        wdec_ref[...], sparse_t.astype(jnp.bfloat16),
        (((1,), (0,)), ((), ())),
        preferred_element_type=jnp.float32,
        precision=jax.lax.Precision.DEFAULT) + bdec_ref[...]
    recon_ref[...] = recon_t.T


def kernel(x, W_enc, b_enc, W_dec, b_dec):
    n = x.shape[0]
    xt = x.T
    wdec_bf16 = W_dec.astype(jnp.bfloat16)
    benc = b_enc.reshape(-1, 1)
    bdec = b_dec.reshape(-1, 1)

    grid = (n // _TM,)
    sparse, recon = pl.pallas_call(
        _fused_body,
        grid=grid,
        in_specs=[
            pl.BlockSpec((_INPUT_DIM, _TM), lambda i: (0, i)),
            pl.BlockSpec((_LATENT_DIM, _INPUT_DIM), lambda i: (0, 0)),
            pl.BlockSpec((_LATENT_DIM, 1), lambda i: (0, 0)),
            pl.BlockSpec((_INPUT_DIM, _LATENT_DIM), lambda i: (0, 0)),
            pl.BlockSpec((_INPUT_DIM, 1), lambda i: (0, 0)),
        ],
        out_specs=[
            pl.BlockSpec((_TM, _LATENT_DIM), lambda i: (i, 0)),
            pl.BlockSpec((_TM, _INPUT_DIM), lambda i: (i, 0)),
        ],
        out_shape=[
            jax.ShapeDtypeStruct((n, _LATENT_DIM), jnp.float32),
            jax.ShapeDtypeStruct((n, _INPUT_DIM), jnp.float32),
        ],
        compiler_params=pltpu.CompilerParams(
            dimension_semantics=("arbitrary",)),
    )(xt, W_enc, benc, wdec_bf16, bdec)
    return (recon, sparse)


# speedup vs baseline: 1.0845x; 1.0845x over previous
import jax
import jax.numpy as jnp
from jax.experimental import pallas as pl
from jax.experimental.pallas import tpu as pltpu

_INPUT_DIM = 768
_LATENT_DIM = 3072
_K = 32
_TM = 512


def _fused_body(x_ref, wenc_ref, benc_ref, wdec_ref, bdec_ref,
                sparse_ref, recon_ref):
    x = x_ref[...]
    latents = jax.lax.dot_general(
        x, wenc_ref[...], (((1,), (0,)), ((), ())),
        preferred_element_type=jnp.float32,
        precision=jax.lax.Precision.DEFAULT) + benc_ref[...]

    bits = jax.lax.bitcast_convert_type(latents, jnp.int32)
    keys = jnp.where(bits < 0, jnp.int32(-2147483648) - bits, bits)

    lo0 = jnp.full((latents.shape[0], 1), -2139095041, jnp.int32)
    hi0 = jnp.full((latents.shape[0], 1), 2139095041, jnp.int32)

    cnt0 = jnp.zeros((latents.shape[0], 1), jnp.int32)

    def bs_body(_, carry):
        lo, hi, cnt_hi = carry
        mid = (lo >> 1) + (hi >> 1) + (lo & hi & 1)
        cnt = jnp.sum((keys > mid).astype(jnp.int32), axis=1, keepdims=True)
        ge_k = cnt >= _K
        lo = jnp.where(ge_k, mid + 1, lo)
        hi = jnp.where(ge_k, hi, mid)
        cnt_hi = jnp.where(ge_k, cnt_hi, cnt)
        return lo, hi, cnt_hi

    lo, _, cnt_gt = jax.lax.fori_loop(0, 32, bs_body, (lo0, hi0, cnt0),
                                      unroll=2)
    vstar = lo

    deficit = _K - cnt_gt
    eq = keys == vstar
    iota = jax.lax.broadcasted_iota(jnp.int32, latents.shape, 1)
    sparse0 = jnp.where(keys > vstar, latents, 0.0)

    def tie_body(_, carry):
        sparse, deficit = carry
        pick = jnp.logical_and(eq, sparse == 0.0)
        cand = jnp.where(pick, iota, _LATENT_DIM)
        amin = jnp.min(cand, axis=1, keepdims=True)
        add = jnp.logical_and(iota == amin, deficit > 0)
        sparse = jnp.where(add, latents, sparse)
        deficit = deficit - (deficit > 0).astype(jnp.int32)
        return sparse, deficit

    sparse, _ = jax.lax.fori_loop(0, 2, tie_body, (sparse0, deficit))
    sparse_ref[...] = sparse
    recon_ref[...] = jax.lax.dot_general(
        sparse.astype(jnp.bfloat16), wdec_ref[...], (((1,), (0,)), ((), ())),
        preferred_element_type=jnp.float32,
        precision=jax.lax.Precision.DEFAULT) + bdec_ref[...]


def kernel(x, W_enc, b_enc, W_dec, b_dec):
    n = x.shape[0]
    wenc_t = W_enc.T
    wdec_t = W_dec.T.astype(jnp.bfloat16)
    benc = b_enc.reshape(1, -1)
    bdec = b_dec.reshape(1, -1)

    grid = (n // _TM,)
    sparse, recon = pl.pallas_call(
        _fused_body,
        grid=grid,
        in_specs=[
            pl.BlockSpec((_TM, _INPUT_DIM), lambda i: (i, 0)),
            pl.BlockSpec((_INPUT_DIM, _LATENT_DIM), lambda i: (0, 0)),
            pl.BlockSpec((1, _LATENT_DIM), lambda i: (0, 0)),
            pl.BlockSpec((_LATENT_DIM, _INPUT_DIM), lambda i: (0, 0)),
            pl.BlockSpec((1, _INPUT_DIM), lambda i: (0, 0)),
        ],
        out_specs=[
            pl.BlockSpec((_TM, _LATENT_DIM), lambda i: (i, 0)),
            pl.BlockSpec((_TM, _INPUT_DIM), lambda i: (i, 0)),
        ],
        out_shape=[
            jax.ShapeDtypeStruct((n, _LATENT_DIM), jnp.float32),
            jax.ShapeDtypeStruct((n, _INPUT_DIM), jnp.float32),
        ],
        compiler_params=pltpu.CompilerParams(
            dimension_semantics=("arbitrary",)),
    )(x, wenc_t, benc, wdec_t, bdec)
    return (recon, sparse)


# speedup vs baseline: 1.1115x; 1.0248x over previous
import jax
import jax.numpy as jnp
from jax.experimental import pallas as pl
from jax.experimental.pallas import tpu as pltpu

_INPUT_DIM = 768
_LATENT_DIM = 3072
_K = 32
_TM = 512


def _fused_body(x_ref, wenc_ref, benc_ref, wdec_ref, bdec_ref,
                sparse_ref, recon_ref):
    x = x_ref[...]
    latents = jax.lax.dot_general(
        x, wenc_ref[...], (((1,), (0,)), ((), ())),
        preferred_element_type=jnp.float32,
        precision=jax.lax.Precision.DEFAULT) + benc_ref[...]

    bits = jax.lax.bitcast_convert_type(latents, jnp.int32)
    keys = jnp.where(bits < 0, jnp.int32(-2147483648) - bits, bits)

    lo0 = jnp.full((latents.shape[0], 1), -2139095041, jnp.int32)
    hi0 = jnp.full((latents.shape[0], 1), 2139095041, jnp.int32)

    cnt0 = jnp.zeros((latents.shape[0], 1), jnp.int32)

    def bs_body(_, carry):
        lo, hi, cnt_hi = carry
        mid = (lo >> 1) + (hi >> 1) + (lo & hi & 1)
        cnt = jnp.sum((keys > mid).astype(jnp.int32), axis=1, keepdims=True)
        ge_k = cnt >= _K
        lo = jnp.where(ge_k, mid + 1, lo)
        hi = jnp.where(ge_k, hi, mid)
        cnt_hi = jnp.where(ge_k, cnt_hi, cnt)
        return lo, hi, cnt_hi

    lo, _, cnt_gt = jax.lax.fori_loop(0, 32, bs_body, (lo0, hi0, cnt0),
                                      unroll=4)
    vstar = lo

    deficit = _K - cnt_gt
    eq = keys == vstar
    iota = jax.lax.broadcasted_iota(jnp.int32, latents.shape, 1)
    sparse0 = jnp.where(keys > vstar, latents, 0.0)

    def tie_body(_, carry):
        sparse, deficit = carry
        pick = jnp.logical_and(eq, sparse == 0.0)
        cand = jnp.where(pick, iota, _LATENT_DIM)
        amin = jnp.min(cand, axis=1, keepdims=True)
        add = jnp.logical_and(iota == amin, deficit > 0)
        sparse = jnp.where(add, latents, sparse)
        deficit = deficit - (deficit > 0).astype(jnp.int32)
        return sparse, deficit

    sparse, _ = jax.lax.fori_loop(0, 2, tie_body, (sparse0, deficit))
    sparse_ref[...] = sparse
    recon_ref[...] = jax.lax.dot_general(
        sparse.astype(jnp.bfloat16), wdec_ref[...], (((1,), (0,)), ((), ())),
        preferred_element_type=jnp.float32,
        precision=jax.lax.Precision.DEFAULT) + bdec_ref[...]


def kernel(x, W_enc, b_enc, W_dec, b_dec):
    n = x.shape[0]
    wenc_t = W_enc.T
    wdec_t = W_dec.T.astype(jnp.bfloat16)
    benc = b_enc.reshape(1, -1)
    bdec = b_dec.reshape(1, -1)

    grid = (n // _TM,)
    sparse, recon = pl.pallas_call(
        _fused_body,
        grid=grid,
        in_specs=[
            pl.BlockSpec((_TM, _INPUT_DIM), lambda i: (i, 0)),
            pl.BlockSpec((_INPUT_DIM, _LATENT_DIM), lambda i: (0, 0)),
            pl.BlockSpec((1, _LATENT_DIM), lambda i: (0, 0)),
            pl.BlockSpec((_LATENT_DIM, _INPUT_DIM), lambda i: (0, 0)),
            pl.BlockSpec((1, _INPUT_DIM), lambda i: (0, 0)),
        ],
        out_specs=[
            pl.BlockSpec((_TM, _LATENT_DIM), lambda i: (i, 0)),
            pl.BlockSpec((_TM, _INPUT_DIM), lambda i: (i, 0)),
        ],
        out_shape=[
            jax.ShapeDtypeStruct((n, _LATENT_DIM), jnp.float32),
            jax.ShapeDtypeStruct((n, _INPUT_DIM), jnp.float32),
        ],
        compiler_params=pltpu.CompilerParams(
            dimension_semantics=("arbitrary",)),
    )(x, wenc_t, benc, wdec_t, bdec)
    return (recon, sparse)
